# baseline (speedup 1.0000x reference)
import functools

import jax
import jax.numpy as jnp
from jax import lax
from jax.experimental import pallas as pl
from jax.experimental.pallas import tpu as pltpu
from jax.experimental.pallas import tpu_sc as plsc

B = 16384
F = 26
ND = 13
NC = 2
NS = 16
NW = NC * NS
BPW = B // NW
CHUNK = 128
NSTAGE = BPW // CHUNK
GPS = CHUNK // 16
TPAD = 1000064


def _body(table_hbm, idx_hbm, dense_hbm, wv_hbm, out_hbm,
          idx_v, vals_v, dense_v, w_v, out_v, sem0, sem1, sem2, sem3):
    wid = lax.axis_index("s") * NC + lax.axis_index("c")
    base = wid * BPW

    sems = (sem0, sem1, sem2, sem3)

    def fire(k):
        sem = sems[k]
        return [
            pltpu.async_copy(
                table_hbm.at[idx_v.at[f, pl.ds(k * CHUNK, CHUNK)]],
                vals_v.at[f, pl.ds(k * CHUNK, CHUNK)],
                sem,
            )
            for f in range(F)
        ]

    pending = []
    for k in range(NSTAGE):
        pltpu.sync_copy(idx_hbm.at[:, pl.ds(base + k * CHUNK, CHUNK)],
                        idx_v.at[:, pl.ds(k * CHUNK, CHUNK)])
        pending.append(fire(k))

    pltpu.sync_copy(dense_hbm.at[:, pl.ds(base, BPW)], dense_v)
    pltpu.sync_copy(wv_hbm, w_v)

    w16 = w_v[...]
    wsplat = [
        jnp.take_along_axis(w16, jnp.full((16,), d, jnp.int32), axis=0)
        for d in range(ND + 1)
    ]

    def compute(k):
        @pl.loop(0, GPS)
        def _compute(gg):
            g = k * GPS + gg
            acc = wsplat[ND]
            for f in range(F):
                acc = acc + vals_v[f, pl.ds(g * 16, 16)]
            for d in range(ND):
                acc = acc + wsplat[d] * dense_v[d, pl.ds(g * 16, 16)]
            out_v[pl.ds(g * 16, 16)] = acc

    for k in range(NSTAGE):
        for c in pending[k]:
            c.wait()
        compute(k)

    pltpu.sync_copy(out_v, out_hbm.at[pl.ds(base, BPW)])


@jax.jit
def _lr_sc(table_flat, idx_t, dense_t, wv):
    mesh = plsc.VectorSubcoreMesh(core_axis_name="c", subcore_axis_name="s")
    return pl.kernel(
        _body,
        out_type=jax.ShapeDtypeStruct((B,), jnp.float32),
        mesh=mesh,
        compiler_params=pltpu.CompilerParams(needs_layout_passes=False),
        scratch_types=[
            pltpu.VMEM((F, BPW), jnp.int32),
            pltpu.VMEM((F, BPW), jnp.float32),
            pltpu.VMEM((ND, BPW), jnp.float32),
            pltpu.VMEM((16,), jnp.float32),
            pltpu.VMEM((BPW,), jnp.float32),
            pltpu.SemaphoreType.DMA,
            pltpu.SemaphoreType.DMA,
            pltpu.SemaphoreType.DMA,
            pltpu.SemaphoreType.DMA,
        ],
    )(table_flat, idx_t, dense_t, wv)


def kernel(dense, sparse, sparse_table, dense_W, dense_b, bias):
    idx_t = sparse.astype(jnp.int32).T
    dense_t = dense.T
    table_flat = sparse_table[jnp.arange(1000000, dtype=jnp.int32), 0]
    wv = jnp.concatenate(
        [dense_W.reshape(-1),
         (dense_b + bias).reshape(-1),
         jnp.zeros(2, jnp.float32)]
    )
    return _lr_sc(table_flat, idx_t, dense_t, wv)


# speedup vs baseline: 1.4377x; 1.4377x over previous
import functools

import jax
import jax.numpy as jnp
from jax import lax
from jax.experimental import pallas as pl
from jax.experimental.pallas import tpu as pltpu
from jax.experimental.pallas import tpu_sc as plsc

B = 16384
F = 26
ND = 13
NC = 2
NS = 16
NW = NC * NS
BPW = B // NW
CHUNK = 128
NSTAGE = BPW // CHUNK
GPS = CHUNK // 16
TPAD = 1000064


def _body(table_hbm, idx_hbm, dense_hbm, wv_hbm, out_hbm,
          idx_v, vals_v, dense_v, w_v, out_v, sem0, sem1, sem2, sem3):
    wid = lax.axis_index("s") * NC + lax.axis_index("c")
    base = wid * BPW

    sems = (sem0, sem1, sem2, sem3)

    def fire(k):
        sem = sems[k]
        return [
            pltpu.async_copy(
                table_hbm.at[idx_v.at[f, pl.ds(k * CHUNK, CHUNK)]],
                vals_v.at[f, pl.ds(k * CHUNK, CHUNK)],
                sem,
            )
            for f in range(F)
        ]

    pending = []
    for k in range(NSTAGE):
        pltpu.sync_copy(idx_hbm.at[:, pl.ds(base + k * CHUNK, CHUNK)],
                        idx_v.at[:, pl.ds(k * CHUNK, CHUNK)])
        pending.append(fire(k))

    pltpu.sync_copy(dense_hbm.at[:, pl.ds(base, BPW)], dense_v)
    pltpu.sync_copy(wv_hbm, w_v)

    w16 = w_v[...]
    wsplat = [
        jnp.take_along_axis(w16, jnp.full((16,), d, jnp.int32), axis=0)
        for d in range(ND + 1)
    ]

    def compute(k):
        @pl.loop(0, GPS)
        def _compute(gg):
            g = k * GPS + gg
            acc = wsplat[ND]
            for f in range(F):
                acc = acc + vals_v[f, pl.ds(g * 16, 16)]
            for d in range(ND):
                acc = acc + wsplat[d] * dense_v[d, pl.ds(g * 16, 16)]
            out_v[pl.ds(g * 16, 16)] = acc

    for k in range(NSTAGE):
        for c in pending[k]:
            c.wait()
        compute(k)

    pltpu.sync_copy(out_v, out_hbm.at[pl.ds(base, BPW)])


@jax.jit
def _lr_sc(table_flat, idx_t, dense_t, wv):
    mesh = plsc.VectorSubcoreMesh(core_axis_name="c", subcore_axis_name="s")
    return pl.kernel(
        _body,
        out_type=jax.ShapeDtypeStruct((B,), jnp.float32),
        mesh=mesh,
        compiler_params=pltpu.CompilerParams(needs_layout_passes=False),
        scratch_types=[
            pltpu.VMEM((F, BPW), jnp.int32),
            pltpu.VMEM((F, BPW), jnp.float32),
            pltpu.VMEM((ND, BPW), jnp.float32),
            pltpu.VMEM((16,), jnp.float32),
            pltpu.VMEM((BPW,), jnp.float32),
            pltpu.SemaphoreType.DMA,
            pltpu.SemaphoreType.DMA,
            pltpu.SemaphoreType.DMA,
            pltpu.SemaphoreType.DMA,
        ],
    )(table_flat, idx_t, dense_t, wv)


def kernel(dense, sparse, sparse_table, dense_W, dense_b, bias):
    idx_t = sparse.astype(jnp.int32).T
    dense_t = dense.T
    table_flat = lax.reshape(sparse_table, (1000000,), dimensions=(1, 0))
    wv = jnp.concatenate(
        [dense_W.reshape(-1),
         (dense_b + bias).reshape(-1),
         jnp.zeros(2, jnp.float32)]
    )
    return _lr_sc(table_flat, idx_t, dense_t, wv)
